# baseline (speedup 1.0000x reference)
import functools

import jax
import jax.numpy as jnp
from jax import lax
from jax.experimental import pallas as pl
from jax.experimental.pallas import tpu as pltpu
from jax.experimental.pallas import tpu_sc as plsc

N = 10000
D = 128
W = D + 16
GW = 2 * D + 16
NP = 10112
CH = 32
NW = 32
RZ = NP // 16


def _z(i):
    return i * 0


def _prep_body(x_ref, w_ref, u_ref, c_ref, gxv_ref, xv_ref):
    xw = jnp.dot(x_ref[...], w_ref[...],
                 preferred_element_type=jnp.float32,
                 precision=lax.Precision.HIGHEST)
    gxv_ref[:, :D] = xw[:, :D] - xw[:, D:]
    gxv_ref[:, D:2 * D] = xw[:, D:]
    uv = u_ref[:, 0:1] - u_ref[:, 1:2]
    xv = jnp.dot(x_ref[...], uv,
                 preferred_element_type=jnp.float32,
                 precision=lax.Precision.HIGHEST)
    xv16 = jnp.broadcast_to(xv, (xv.shape[0], 16))
    cd = c_ref[0, 0] - c_ref[0, 1]
    gxv_ref[:, 2 * D:] = xv16 + cd
    xv_ref[...] = xv16


def _prep(x, weight, u, c2):
    R = 1000
    return pl.pallas_call(
        _prep_body,
        grid=(N // R,),
        in_specs=[
            pl.BlockSpec((R, D), lambda i: (i, _z(i))),
            pl.BlockSpec((D, 2 * D), lambda i: (_z(i), _z(i))),
            pl.BlockSpec((D, 2), lambda i: (_z(i), _z(i))),
            pl.BlockSpec((1, 2), lambda i: (_z(i), _z(i))),
        ],
        out_specs=[
            pl.BlockSpec((R, GW), lambda i: (i, _z(i))),
            pl.BlockSpec((R, 16), lambda i: (i, _z(i))),
        ],
        out_shape=[
            jax.ShapeDtypeStruct((N, GW), jnp.float32),
            jax.ShapeDtypeStruct((N, 16), jnp.float32),
        ],
    )(x, weight, u, c2)


def _sc_body(nch, gxv_hbm, xv_hbm, eidx_hbm, z_hbm,
             acc_out,
             idx0, idx1, idx2, dstm0, dstm1, rows0, rows1, rows2,
             xvd0, xvd1, xvd2, msg0, msg1, qbuf, acc_sh,
             semi0, semi1, semi2, semg0, semg1, semg2, sems0, sems1):
    i32 = jnp.int32
    c_id = lax.axis_index("c")
    s_id = lax.axis_index("s")
    wid = c_id * i32(16) + s_id

    idx = (idx0, idx1, idx2)
    dstm = (dstm0, dstm1)
    rows = (rows0, rows1, rows2)
    xvd = (xvd0, xvd1, xvd2)
    msg = (msg0, msg1)
    semi = (semi0, semi1, semi2)
    semg = (semg0, semg1, semg2)
    sems = (sems0, sems1)

    zb = s_id * i32(RZ)
    pltpu.sync_copy(z_hbm.at[pl.ds(zb, RZ)], acc_sh.at[pl.ds(zb, RZ)])
    ones16 = jnp.ones((16,), jnp.float32)
    for p in (0, 1):
        for r in range(CH):
            msg[p][i32(r), pl.ds(D, 16)] = ones16
    plsc.subcore_barrier()

    cbase = wid * i32(nch)
    zeros16i = jnp.zeros((16,), jnp.int32)
    c_xv = jnp.full((16,), 2 * D, jnp.int32)

    def idx_copy(ci, p):
        return pltpu.make_async_copy(
            eidx_hbm.at[pl.ds((cbase + ci) * i32(2 * CH), 2 * CH)],
            idx[p], semi[p])

    def rows_copy(p):
        return pltpu.make_async_copy(
            gxv_hbm.at[idx[p].at[pl.ds(0, CH)]], rows[p], semg[p])

    def xvd_copy(p):
        return pltpu.make_async_copy(
            xv_hbm.at[idx[p].at[pl.ds(CH, CH)]], xvd[p], semg[p])

    def scat_start(p):
        pltpu.async_copy(msg[p], acc_sh.at[dstm[p]], sems[p], add=True)

    def scat_wait(p):
        pltpu.make_async_copy(msg[p], acc_sh.at[dstm[p]], sems[p]).wait()

    def start_gathers(p):
        rows_copy(p).start()
        xvd_copy(p).start()

    def wait_gathers(p):
        rows_copy(p).wait()
        xvd_copy(p).wait()

    def compute(p3, p2):
        for g in range(CH // 16):
            rb = i32(g * 16)
            rows16 = rb + lax.iota(jnp.int32, 16)
            xvs = plsc.load_gather(rows[p3], [rows16, c_xv])
            xvdv = plsc.load_gather(xvd[p3], [rows16, zeros16i])
            q = 1.0 / (1.0 + jnp.exp(-(xvs - xvdv)))
            qbuf[pl.ds(rb, 16)] = q
            srcv = idx[p3][pl.ds(i32(g * 16), 16)]
            dstv = idx[p3][pl.ds(i32(CH + g * 16), 16)]
            dstm[p2][pl.ds(rb, 16)] = jnp.where(srcv != dstv, dstv, i32(N))

        @plsc.parallel_loop(jnp.int32(0), jnp.int32(CH), jnp.int32(1), unroll=2)
        def _(r):
            qv = plsc.load_gather(qbuf, [zeros16i + r])
            for k in range(D // 16):
                col = k * 16
                gseg = rows[p3][r, pl.ds(col, 16)]
                bseg = rows[p3][r, pl.ds(D + col, 16)]
                msg[p2][r, pl.ds(col, 16)] = bseg + qv * gseg

    idx_copy(i32(0), 0).start()
    idx_copy(i32(1), 1).start()
    idx_copy(i32(0), 0).wait()
    start_gathers(0)
    idx_copy(i32(1), 1).wait()
    start_gathers(1)
    idx_copy(i32(2), 2).start()

    def six(i6, carry):
        b = i6 * i32(6)
        for j in range(6):
            p3 = j % 3
            p2 = j % 2
            p3n2 = (j + 2) % 3
            t = b + i32(j)
            wait_gathers(p3)
            idx_copy(i32(0), p3n2).wait()
            start_gathers(p3n2)

            if j < 2:
                cond = b + i32(j) >= i32(2)

                @pl.when(cond)
                def _(p2=p2):
                    scat_wait(p2)
            else:
                scat_wait(p2)

            compute(p3, p2)
            idx_copy(t + i32(3), p3).start()
            scat_start(p2)
        return carry

    lax.fori_loop(jnp.int32(0), jnp.int32(nch // 6), six, 0)

    wait_gathers(0)
    wait_gathers(1)
    idx_copy(i32(0), 2).wait()
    scat_wait(0)
    scat_wait(1)

    plsc.subcore_barrier()
    pltpu.sync_copy(acc_sh.at[pl.ds(zb, RZ)], acc_out.at[c_id, pl.ds(zb, RZ)])


def _scatter_stage(gxv, xv16, eidx, zrows, nch):
    mesh = plsc.VectorSubcoreMesh(core_axis_name="c", subcore_axis_name="s")
    kfn = functools.partial(
        pl.kernel,
        out_type=jax.ShapeDtypeStruct((2, NP, W), jnp.float32),
        mesh=mesh,
        scratch_types=[
            pltpu.VMEM((2 * CH,), jnp.int32),
            pltpu.VMEM((2 * CH,), jnp.int32),
            pltpu.VMEM((2 * CH,), jnp.int32),
            pltpu.VMEM((CH,), jnp.int32),
            pltpu.VMEM((CH,), jnp.int32),
            pltpu.VMEM((CH, GW), jnp.float32),
            pltpu.VMEM((CH, GW), jnp.float32),
            pltpu.VMEM((CH, GW), jnp.float32),
            pltpu.VMEM((CH, 16), jnp.float32),
            pltpu.VMEM((CH, 16), jnp.float32),
            pltpu.VMEM((CH, 16), jnp.float32),
            pltpu.VMEM((CH, W), jnp.float32),
            pltpu.VMEM((CH, W), jnp.float32),
            pltpu.VMEM((CH,), jnp.float32),
            pltpu.VMEM_SHARED((NP, W), jnp.float32),
            pltpu.SemaphoreType.DMA,
            pltpu.SemaphoreType.DMA,
            pltpu.SemaphoreType.DMA,
            pltpu.SemaphoreType.DMA,
            pltpu.SemaphoreType.DMA,
            pltpu.SemaphoreType.DMA,
            pltpu.SemaphoreType.DMA,
            pltpu.SemaphoreType.DMA,
        ],
        compiler_params=pltpu.CompilerParams(
            needs_layout_passes=False, use_tc_tiling_on_sc=False),
    )(functools.partial(_sc_body, nch))
    return kfn(gxv, xv16, eidx, zrows)


def _fin_body(x_ref, gxv_ref, acc_ref, bias_ref, c_ref, o_ref):
    cd = c_ref[0, 0] - c_ref[0, 1]
    s0 = 1.0 / (1.0 + jnp.exp(-cd))
    self_msg = gxv_ref[:, D:2 * D] + s0 * gxv_ref[:, :D]
    summed = acc_ref[0, :, :D] + acc_ref[1, :, :D] + self_msg
    cnt = 1.0 + acc_ref[0, :, D:D + 1] + acc_ref[1, :, D:D + 1]
    conv = summed / cnt + bias_ref[0]
    o_ref[...] = x_ref[...] + jnp.maximum(conv, 0.0)


def _finalize(x, gxv, acc, bias, c2):
    R = 1024
    return pl.pallas_call(
        _fin_body,
        grid=(-(-N // R),),
        in_specs=[
            pl.BlockSpec((R, D), lambda i: (i, _z(i))),
            pl.BlockSpec((R, GW), lambda i: (i, _z(i))),
            pl.BlockSpec((2, R, W), lambda i: (_z(i), i, _z(i))),
            pl.BlockSpec((1, D), lambda i: (_z(i), _z(i))),
            pl.BlockSpec((1, 2), lambda i: (_z(i), _z(i))),
        ],
        out_specs=pl.BlockSpec((R, D), lambda i: (i, _z(i))),
        out_shape=jax.ShapeDtypeStruct((N, D), jnp.float32),
    )(x, gxv, acc, bias, c2)


def kernel(x, edge_index, weight, u, c, bias):
    E = edge_index.shape[1]
    src = edge_index[0].astype(jnp.int32)
    dst = edge_index[1].astype(jnp.int32)
    nch = -(-E // (NW * CH))
    if nch % 6:
        nch += 6 - nch % 6
    ept = nch * CH
    pad = ept * NW - E
    if pad:
        src = jnp.concatenate([src, jnp.zeros((pad,), jnp.int32)])
        dst = jnp.concatenate([dst, jnp.zeros((pad,), jnp.int32)])
    eidx = jnp.stack([src.reshape(-1, CH), dst.reshape(-1, CH)],
                     axis=1).reshape(-1)
    eidx = jnp.concatenate([eidx, jnp.zeros((8 * CH,), jnp.int32)])

    c2 = jnp.reshape(c, (1, 2)).astype(jnp.float32)
    gxv, xv16 = _prep(x, weight, u, c2)
    zrows = jnp.zeros((NP, W), jnp.float32)
    acc = _scatter_stage(gxv, xv16, eidx, zrows, nch)
    return _finalize(x, gxv, acc,
                     jnp.reshape(bias, (1, D)).astype(jnp.float32), c2)


# speedup vs baseline: 1.2028x; 1.2028x over previous
import functools

import jax
import jax.numpy as jnp
from jax import lax
from jax.experimental import pallas as pl
from jax.experimental.pallas import tpu as pltpu
from jax.experimental.pallas import tpu_sc as plsc

N = 10000
D = 128
W = D + 16
GW = 2 * D + 16
NP = 10112
CH = 32
NW = 32
RZ = NP // 16


def _z(i):
    return i * 0


def _prep_body(x_ref, w_ref, u_ref, c_ref, gxv_ref, xv_ref):
    xw = jnp.dot(x_ref[...], w_ref[...],
                 preferred_element_type=jnp.float32,
                 precision=lax.Precision.HIGHEST)
    gxv_ref[:, :D] = xw[:, :D] - xw[:, D:]
    gxv_ref[:, D:2 * D] = xw[:, D:]
    uv = u_ref[:, 0:1] - u_ref[:, 1:2]
    xv = jnp.dot(x_ref[...], uv,
                 preferred_element_type=jnp.float32,
                 precision=lax.Precision.HIGHEST)
    xv16 = jnp.broadcast_to(xv, (xv.shape[0], 16))
    cd = c_ref[0, 0] - c_ref[0, 1]
    gxv_ref[:, 2 * D:] = xv16 + cd
    xv_ref[...] = xv16


def _prep(x, weight, u, c2):
    R = 1000
    return pl.pallas_call(
        _prep_body,
        grid=(N // R,),
        in_specs=[
            pl.BlockSpec((R, D), lambda i: (i, _z(i))),
            pl.BlockSpec((D, 2 * D), lambda i: (_z(i), _z(i))),
            pl.BlockSpec((D, 2), lambda i: (_z(i), _z(i))),
            pl.BlockSpec((1, 2), lambda i: (_z(i), _z(i))),
        ],
        out_specs=[
            pl.BlockSpec((R, GW), lambda i: (i, _z(i))),
            pl.BlockSpec((R, 16), lambda i: (i, _z(i))),
        ],
        out_shape=[
            jax.ShapeDtypeStruct((N, GW), jnp.float32),
            jax.ShapeDtypeStruct((N, 16), jnp.float32),
        ],
    )(x, weight, u, c2)


def _sc_body(nch, gxv_hbm, xv_hbm, eidx_hbm, z_hbm,
             acc_out,
             idx0, idx1, dstm0, dstm1, rows0, rows1, xvd0, xvd1,
             msg0, msg1, acc_sh,
             semi0, semi1, semg0, semg1, sems0, sems1):
    i32 = jnp.int32
    c_id = lax.axis_index("c")
    s_id = lax.axis_index("s")
    wid = c_id * i32(16) + s_id

    idx = (idx0, idx1)
    dstm = (dstm0, dstm1)
    rows = (rows0, rows1)
    xvd = (xvd0, xvd1)
    msg = (msg0, msg1)
    semi = (semi0, semi1)
    semg = (semg0, semg1)
    sems = (sems0, sems1)

    zb = s_id * i32(RZ)
    pltpu.sync_copy(z_hbm.at[pl.ds(zb, RZ)], acc_sh.at[pl.ds(zb, RZ)])
    ones16 = jnp.ones((16,), jnp.float32)
    for p in (0, 1):
        for r in range(CH):
            msg[p][i32(r), pl.ds(D, 16)] = ones16
    plsc.subcore_barrier()

    cbase = wid * i32(nch)
    zeros16i = jnp.zeros((16,), jnp.int32)
    c_xv = jnp.full((16,), 2 * D, jnp.int32)

    def idx_copy(ci, p):
        return pltpu.make_async_copy(
            eidx_hbm.at[pl.ds((cbase + ci) * i32(2 * CH), 2 * CH)],
            idx[p], semi[p])

    def rows_copy(p):
        return pltpu.make_async_copy(
            gxv_hbm.at[idx[p].at[pl.ds(0, CH)]], rows[p], semg[p])

    def xvd_copy(p):
        return pltpu.make_async_copy(
            xv_hbm.at[idx[p].at[pl.ds(CH, CH)]], xvd[p], semg[p])

    def scat_start(p):
        pltpu.async_copy(msg[p], acc_sh.at[dstm[p]], sems[p], add=True)

    def scat_wait(p):
        pltpu.make_async_copy(msg[p], acc_sh.at[dstm[p]], sems[p]).wait()

    def start_gathers(p):
        rows_copy(p).start()
        xvd_copy(p).start()

    def wait_gathers(p):
        rows_copy(p).wait()
        xvd_copy(p).wait()

    def compute(p):
        for g in range(CH // 16):
            rows16 = i32(g * 16) + lax.iota(jnp.int32, 16)
            xvs = plsc.load_gather(rows[p], [rows16, c_xv])
            xvdv = plsc.load_gather(xvd[p], [rows16, zeros16i])
            q = 1.0 / (1.0 + jnp.exp(-(xvs - xvdv)))
            srcv = idx[p][pl.ds(g * 16, 16)]
            dstv = idx[p][pl.ds(CH + g * 16, 16)]
            dstm[p][pl.ds(g * 16, 16)] = jnp.where(srcv != dstv, dstv, i32(N))
            for e in range(16):
                qe = q[e]
                r = g * 16 + e
                for k in range(D // 16):
                    col = k * 16
                    gseg = rows[p][r, pl.ds(col, 16)]
                    bseg = rows[p][r, pl.ds(D + col, 16)]
                    msg[p][r, pl.ds(col, 16)] = bseg + qe * gseg

    idx_copy(i32(0), 0).start()
    idx_copy(i32(0), 0).wait()
    start_gathers(0)
    idx_copy(i32(1), 1).start()

    def pair(i2, carry):
        a = i2 * i32(2)
        wait_gathers(0)

        @pl.when(a >= i32(2))
        def _():
            scat_wait(0)

        idx_copy(i32(0), 1).wait()
        start_gathers(1)
        compute(0)
        idx_copy(a + i32(2), 0).start()
        scat_start(0)

        wait_gathers(1)

        @pl.when(a >= i32(1))
        def _():
            scat_wait(1)

        idx_copy(i32(0), 0).wait()
        start_gathers(0)
        compute(1)
        idx_copy(a + i32(3), 1).start()
        scat_start(1)
        return carry

    lax.fori_loop(jnp.int32(0), jnp.int32(nch // 2), pair, 0)

    idx_copy(i32(0), 1).wait()
    wait_gathers(0)
    scat_wait(0)
    scat_wait(1)

    plsc.subcore_barrier()
    pltpu.sync_copy(acc_sh.at[pl.ds(zb, RZ)], acc_out.at[c_id, pl.ds(zb, RZ)])


def _scatter_stage(gxv, xv16, eidx, zrows, nch):
    mesh = plsc.VectorSubcoreMesh(core_axis_name="c", subcore_axis_name="s")
    kfn = functools.partial(
        pl.kernel,
        out_type=jax.ShapeDtypeStruct((2, NP, W), jnp.float32),
        mesh=mesh,
        scratch_types=[
            pltpu.VMEM((2 * CH,), jnp.int32),
            pltpu.VMEM((2 * CH,), jnp.int32),
            pltpu.VMEM((CH,), jnp.int32),
            pltpu.VMEM((CH,), jnp.int32),
            pltpu.VMEM((CH, GW), jnp.float32),
            pltpu.VMEM((CH, GW), jnp.float32),
            pltpu.VMEM((CH, 16), jnp.float32),
            pltpu.VMEM((CH, 16), jnp.float32),
            pltpu.VMEM((CH, W), jnp.float32),
            pltpu.VMEM((CH, W), jnp.float32),
            pltpu.VMEM_SHARED((NP, W), jnp.float32),
            pltpu.SemaphoreType.DMA,
            pltpu.SemaphoreType.DMA,
            pltpu.SemaphoreType.DMA,
            pltpu.SemaphoreType.DMA,
            pltpu.SemaphoreType.DMA,
            pltpu.SemaphoreType.DMA,
        ],
        compiler_params=pltpu.CompilerParams(
            needs_layout_passes=False, use_tc_tiling_on_sc=False),
    )(functools.partial(_sc_body, nch))
    return kfn(gxv, xv16, eidx, zrows)


def _fin_body(x_ref, gxv_ref, acc_ref, bias_ref, c_ref, o_ref):
    cd = c_ref[0, 0] - c_ref[0, 1]
    s0 = 1.0 / (1.0 + jnp.exp(-cd))
    self_msg = gxv_ref[:, D:2 * D] + s0 * gxv_ref[:, :D]
    summed = acc_ref[0, :, :D] + acc_ref[1, :, :D] + self_msg
    cnt = 1.0 + acc_ref[0, :, D:D + 1] + acc_ref[1, :, D:D + 1]
    conv = summed / cnt + bias_ref[0]
    o_ref[...] = x_ref[...] + jnp.maximum(conv, 0.0)


def _finalize(x, gxv, acc, bias, c2):
    R = 1024
    return pl.pallas_call(
        _fin_body,
        grid=(-(-N // R),),
        in_specs=[
            pl.BlockSpec((R, D), lambda i: (i, _z(i))),
            pl.BlockSpec((R, GW), lambda i: (i, _z(i))),
            pl.BlockSpec((2, R, W), lambda i: (_z(i), i, _z(i))),
            pl.BlockSpec((1, D), lambda i: (_z(i), _z(i))),
            pl.BlockSpec((1, 2), lambda i: (_z(i), _z(i))),
        ],
        out_specs=pl.BlockSpec((R, D), lambda i: (i, _z(i))),
        out_shape=jax.ShapeDtypeStruct((N, D), jnp.float32),
    )(x, gxv, acc, bias, c2)


def kernel(x, edge_index, weight, u, c, bias):
    E = edge_index.shape[1]
    src = edge_index[0].astype(jnp.int32)
    dst = edge_index[1].astype(jnp.int32)
    nch = -(-E // (NW * CH))
    if nch % 2:
        nch += 1
    ept = nch * CH
    pad = ept * NW - E
    if pad:
        src = jnp.concatenate([src, jnp.zeros((pad,), jnp.int32)])
        dst = jnp.concatenate([dst, jnp.zeros((pad,), jnp.int32)])
    eidx = jnp.stack([src.reshape(-1, CH), dst.reshape(-1, CH)],
                     axis=1).reshape(-1)
    eidx = jnp.concatenate([eidx, jnp.zeros((4 * CH,), jnp.int32)])

    c2 = jnp.reshape(c, (1, 2)).astype(jnp.float32)
    gxv, xv16 = _prep(x, weight, u, c2)
    zrows = jnp.zeros((NP, W), jnp.float32)
    acc = _scatter_stage(gxv, xv16, eidx, zrows, nch)
    return _finalize(x, gxv, acc,
                     jnp.reshape(bias, (1, D)).astype(jnp.float32), c2)


# speedup vs baseline: 1.3240x; 1.1007x over previous
import functools

import jax
import jax.numpy as jnp
from jax import lax
from jax.experimental import pallas as pl
from jax.experimental.pallas import tpu as pltpu
from jax.experimental.pallas import tpu_sc as plsc

N = 10000
D = 128
W = D + 16
GW = 2 * D
NXV = 10016
NP = 10112
CH = 32
NW = 32
RZ = NP // 16


def _z(i):
    return i * 0


def _prep_body(x_ref, w_ref, u_ref, gxv_ref, xv_ref):
    xw = jnp.dot(x_ref[...], w_ref[...],
                 preferred_element_type=jnp.float32,
                 precision=lax.Precision.HIGHEST)
    gxv_ref[:, :D] = xw[:, :D] - xw[:, D:]
    gxv_ref[:, D:2 * D] = xw[:, D:]
    uv = u_ref[:, 0:1] - u_ref[:, 1:2]
    xv = jnp.dot(x_ref[...], uv,
                 preferred_element_type=jnp.float32,
                 precision=lax.Precision.HIGHEST)
    xv_ref[...] = xv


def _prep(x, weight, u):
    R = 1000
    return pl.pallas_call(
        _prep_body,
        grid=(N // R,),
        in_specs=[
            pl.BlockSpec((R, D), lambda i: (i, _z(i))),
            pl.BlockSpec((D, 2 * D), lambda i: (_z(i), _z(i))),
            pl.BlockSpec((D, 2), lambda i: (_z(i), _z(i))),
        ],
        out_specs=[
            pl.BlockSpec((R, GW), lambda i: (i, _z(i))),
            pl.BlockSpec((R, 1), lambda i: (i, _z(i))),
        ],
        out_shape=[
            jax.ShapeDtypeStruct((N, GW), jnp.float32),
            jax.ShapeDtypeStruct((N, 1), jnp.float32),
        ],
    )(x, weight, u)


def _sc_body(nch, gxv_hbm, xv_hbm, eidx_hbm, cd_hbm, z_hbm,
             acc_out,
             idx0, idx1, dstm0, dstm1, rows0, rows1,
             msg0, msg1, xvl, cd_v, acc_sh,
             semi0, semi1, semg0, semg1, sems0, sems1):
    i32 = jnp.int32
    c_id = lax.axis_index("c")
    s_id = lax.axis_index("s")
    wid = c_id * i32(16) + s_id

    idx = (idx0, idx1)
    dstm = (dstm0, dstm1)
    rows = (rows0, rows1)
    msg = (msg0, msg1)
    semi = (semi0, semi1)
    semg = (semg0, semg1)
    sems = (sems0, sems1)

    zb = s_id * i32(RZ)
    pltpu.sync_copy(z_hbm.at[pl.ds(zb, RZ)], acc_sh.at[pl.ds(zb, RZ)])
    pltpu.sync_copy(xv_hbm, xvl)
    pltpu.sync_copy(cd_hbm, cd_v)
    ones16 = jnp.ones((16,), jnp.float32)
    for p in (0, 1):
        for r in range(CH):
            msg[p][i32(r), pl.ds(D, 16)] = ones16
    plsc.subcore_barrier()

    cbase = wid * i32(nch)

    def idx_copy(ci, p):
        return pltpu.make_async_copy(
            eidx_hbm.at[pl.ds((cbase + ci) * i32(2 * CH), 2 * CH)],
            idx[p], semi[p])

    def rows_copy(p):
        return pltpu.make_async_copy(
            gxv_hbm.at[idx[p].at[pl.ds(0, CH)]], rows[p], semg[p])

    def scat_start(p):
        pltpu.async_copy(msg[p], acc_sh.at[dstm[p]], sems[p], add=True)

    def scat_wait(p):
        pltpu.make_async_copy(msg[p], acc_sh.at[dstm[p]], sems[p]).wait()

    def start_gathers(p):
        rows_copy(p).start()

    def wait_gathers(p):
        rows_copy(p).wait()

    def compute(p):
        cdv = cd_v[...]
        for g in range(CH // 16):
            srcv = idx[p][pl.ds(g * 16, 16)]
            dstv = idx[p][pl.ds(CH + g * 16, 16)]
            xvs = plsc.load_gather(xvl, [srcv])
            xvdv = plsc.load_gather(xvl, [dstv])
            q = 1.0 / (1.0 + jnp.exp(-(xvs - xvdv + cdv)))
            dstm[p][pl.ds(g * 16, 16)] = jnp.where(srcv != dstv, dstv, i32(N))
            for e in range(16):
                qe = q[e]
                r = g * 16 + e
                for k in range(D // 16):
                    col = k * 16
                    gseg = rows[p][r, pl.ds(col, 16)]
                    bseg = rows[p][r, pl.ds(D + col, 16)]
                    msg[p][r, pl.ds(col, 16)] = bseg + qe * gseg

    idx_copy(i32(0), 0).start()
    idx_copy(i32(0), 0).wait()
    start_gathers(0)
    idx_copy(i32(1), 1).start()

    def pair(i2, carry):
        a = i2 * i32(2)
        wait_gathers(0)

        @pl.when(a >= i32(2))
        def _():
            scat_wait(0)

        idx_copy(i32(0), 1).wait()
        start_gathers(1)
        compute(0)
        idx_copy(a + i32(2), 0).start()
        scat_start(0)

        wait_gathers(1)

        @pl.when(a >= i32(1))
        def _():
            scat_wait(1)

        idx_copy(i32(0), 0).wait()
        start_gathers(0)
        compute(1)
        idx_copy(a + i32(3), 1).start()
        scat_start(1)
        return carry

    lax.fori_loop(jnp.int32(0), jnp.int32(nch // 2), pair, 0)

    idx_copy(i32(0), 1).wait()
    wait_gathers(0)
    scat_wait(0)
    scat_wait(1)

    plsc.subcore_barrier()
    pltpu.sync_copy(acc_sh.at[pl.ds(zb, RZ)], acc_out.at[c_id, pl.ds(zb, RZ)])


def _scatter_stage(gxv, xv1, eidx, cd16, zrows, nch):
    mesh = plsc.VectorSubcoreMesh(core_axis_name="c", subcore_axis_name="s")
    kfn = functools.partial(
        pl.kernel,
        out_type=jax.ShapeDtypeStruct((2, NP, W), jnp.float32),
        mesh=mesh,
        scratch_types=[
            pltpu.VMEM((2 * CH,), jnp.int32),
            pltpu.VMEM((2 * CH,), jnp.int32),
            pltpu.VMEM((CH,), jnp.int32),
            pltpu.VMEM((CH,), jnp.int32),
            pltpu.VMEM((CH, GW), jnp.float32),
            pltpu.VMEM((CH, GW), jnp.float32),
            pltpu.VMEM((CH, W), jnp.float32),
            pltpu.VMEM((CH, W), jnp.float32),
            pltpu.VMEM((NXV,), jnp.float32),
            pltpu.VMEM((16,), jnp.float32),
            pltpu.VMEM_SHARED((NP, W), jnp.float32),
            pltpu.SemaphoreType.DMA,
            pltpu.SemaphoreType.DMA,
            pltpu.SemaphoreType.DMA,
            pltpu.SemaphoreType.DMA,
            pltpu.SemaphoreType.DMA,
            pltpu.SemaphoreType.DMA,
        ],
        compiler_params=pltpu.CompilerParams(
            needs_layout_passes=False, use_tc_tiling_on_sc=False),
    )(functools.partial(_sc_body, nch))
    return kfn(gxv, xv1, eidx, cd16, zrows)


def _fin_body(x_ref, gxv_ref, acc_ref, bias_ref, c_ref, o_ref):
    cd = c_ref[0, 0] - c_ref[0, 1]
    s0 = 1.0 / (1.0 + jnp.exp(-cd))
    self_msg = gxv_ref[:, D:2 * D] + s0 * gxv_ref[:, :D]
    summed = acc_ref[0, :, :D] + acc_ref[1, :, :D] + self_msg
    cnt = 1.0 + acc_ref[0, :, D:D + 1] + acc_ref[1, :, D:D + 1]
    conv = summed / cnt + bias_ref[0]
    o_ref[...] = x_ref[...] + jnp.maximum(conv, 0.0)


def _finalize(x, gxv, acc, bias, c2):
    R = 1024
    return pl.pallas_call(
        _fin_body,
        grid=(-(-N // R),),
        in_specs=[
            pl.BlockSpec((R, D), lambda i: (i, _z(i))),
            pl.BlockSpec((R, GW), lambda i: (i, _z(i))),
            pl.BlockSpec((2, R, W), lambda i: (_z(i), i, _z(i))),
            pl.BlockSpec((1, D), lambda i: (_z(i), _z(i))),
            pl.BlockSpec((1, 2), lambda i: (_z(i), _z(i))),
        ],
        out_specs=pl.BlockSpec((R, D), lambda i: (i, _z(i))),
        out_shape=jax.ShapeDtypeStruct((N, D), jnp.float32),
    )(x, gxv, acc, bias, c2)


def kernel(x, edge_index, weight, u, c, bias):
    E = edge_index.shape[1]
    src = edge_index[0].astype(jnp.int32)
    dst = edge_index[1].astype(jnp.int32)
    nch = -(-E // (NW * CH))
    if nch % 2:
        nch += 1
    ept = nch * CH
    pad = ept * NW - E
    if pad:
        src = jnp.concatenate([src, jnp.zeros((pad,), jnp.int32)])
        dst = jnp.concatenate([dst, jnp.zeros((pad,), jnp.int32)])
    eidx = jnp.stack([src.reshape(-1, CH), dst.reshape(-1, CH)],
                     axis=1).reshape(-1)
    eidx = jnp.concatenate([eidx, jnp.zeros((4 * CH,), jnp.int32)])

    c2 = jnp.reshape(c, (1, 2)).astype(jnp.float32)
    gxv, xv1 = _prep(x, weight, u)
    xvp = jnp.pad(jnp.reshape(xv1, (N,)), (0, NXV - N))
    cd16 = jnp.broadcast_to(jnp.reshape(c[0] - c[1], (1,)), (16,)).astype(jnp.float32)
    zrows = jnp.zeros((NP, W), jnp.float32)
    acc = _scatter_stage(gxv, xvp, eidx, cd16, zrows, nch)
    return _finalize(x, gxv, acc,
                     jnp.reshape(bias, (1, D)).astype(jnp.float32), c2)
